# Initial kernel scaffold; baseline (speedup 1.0000x reference)
import jax
import jax.numpy as jnp
from jax.experimental import pallas as pl


def kernel(x, convW0, bn_g0, bn_b0, convW1, bn_g1, bn_b1, convW2, bn_g2, bn_b2, convW3, bn_g3, bn_b3, W5, bn5_g, bn5_b, W1, bn6_g, bn6_b, W2, b2, bn7_g, bn7_b, W3, b3):
    raise NotImplementedError("write your pallas kernel here")


# speedup vs baseline: 11.7383x; 11.7383x over previous
import functools
import jax
import jax.numpy as jnp
import numpy as np
from jax.experimental import pallas as pl
from jax.experimental.pallas import tpu as pltpu

_N = 1024
_B = 32
_K = 20
_RS = np.sqrt(np.float32(1.0 + 1e-5), dtype=np.float32)


def _lrelu(v):
    return jnp.where(v > 0, v, 0.2 * v)


def _gather_cols(src_ref, rows, jstar, out_dtype=None):
    lo = jstar & 127
    hi = jstar >> 7
    parts = []
    for a in range(8):
        sl = slice(a * 128, (a + 1) * 128)
        lo_a = jnp.broadcast_to(lo[None, sl], (rows, 128))
        hi_a = hi[None, sl]
        acc = None
        for t in range(8):
            src = src_ref[:, t * 128:(t + 1) * 128]
            gt = jnp.take_along_axis(src, lo_a, axis=1,
                                     mode="promise_in_bounds")
            acc = gt if acc is None else jnp.where(hi_a == t, gt, acc)
        parts.append(acc)
    return jnp.concatenate(parts, axis=1)


def _edge_layer_body(x_ref, wb_ref, g_ref, b_ref, o_ref, pd_ref, gf_ref,
                     mx_ref, mn_ref, *, cin, cout, cpad):
    x = x_ref[0]
    xb = x.astype(jnp.bfloat16)
    g = jax.lax.dot_general(xb, xb, (((0,), (0,)), ((), ())),
                            preferred_element_type=jnp.float32)
    inner = -2.0 * g
    sq = jnp.sum(x * x, axis=0, keepdims=True)
    sq_col = jnp.transpose(sq)
    pd_ref[...] = ((-sq) - inner) - sq_col

    if cpad == cin:
        gf_ref[cin:, :] = xb

    mx_ref[...] = jnp.full((cout, _N), -jnp.inf, jnp.float32)
    mn_ref[...] = jnp.full((cout, _N), jnp.inf, jnp.float32)

    def body(_, carry):
        kk = pd_ref[...]
        m = jnp.max(kk, axis=0)
        jidx = jax.lax.broadcasted_iota(jnp.int32, (_N, _N), 0)
        jstar = jnp.min(jnp.where(kk == m[None, :], jidx, _N), axis=0)
        pd_ref[...] = jnp.where(jidx == jstar[None, :], -jnp.inf, kk)
        xg = _gather_cols(x_ref.at[0], cin, jstar)
        fk = (xg - x).astype(jnp.bfloat16)
        if cpad == cin:
            gf_ref[:cin, :] = fk
            gf = gf_ref[...]
        else:
            gf = jnp.concatenate(
                [fk, xb,
                 jnp.zeros((2 * (cpad - cin), _N), jnp.bfloat16)], axis=0)
        yk = jax.lax.dot_general(wb_ref[...], gf,
                                 (((1,), (0,)), ((), ())),
                                 preferred_element_type=jnp.float32)
        mx_ref[...] = jnp.maximum(mx_ref[...], yk)
        mn_ref[...] = jnp.minimum(mn_ref[...], yk)
        return carry

    jax.lax.fori_loop(0, _K, body, 0)

    gv = g_ref[...]
    sel = jnp.where(gv >= 0, mx_ref[...], mn_ref[...])
    o_ref[0] = _lrelu(sel / _RS * gv + b_ref[...])


def _edge_layer(x, wb, g, b, cin, cout, cpad):
    body = functools.partial(_edge_layer_body, cin=cin, cout=cout, cpad=cpad)
    return pl.pallas_call(
        body,
        grid=(_B,),
        in_specs=[
            pl.BlockSpec((1, cin, _N), lambda bb: (bb, 0, 0)),
            pl.BlockSpec((cout, 2 * cpad), lambda bb: (0, 0)),
            pl.BlockSpec((cout, 1), lambda bb: (0, 0)),
            pl.BlockSpec((cout, 1), lambda bb: (0, 0)),
        ],
        out_specs=pl.BlockSpec((1, cout, _N), lambda bb: (bb, 0, 0)),
        out_shape=jax.ShapeDtypeStruct((_B, cout, _N), jnp.float32),
        scratch_shapes=[
            pltpu.VMEM((_N, _N), jnp.float32),
            pltpu.VMEM((2 * cpad, _N), jnp.bfloat16),
            pltpu.VMEM((cout, _N), jnp.float32),
            pltpu.VMEM((cout, _N), jnp.float32),
        ],
        compiler_params=pltpu.CompilerParams(
            dimension_semantics=("arbitrary",)),
    )(x, wb, g, b)


def _final_conv_body(f1_ref, f2_ref, f3_ref, f4_ref, w5_ref, g5_ref, b5_ref,
                     o1_ref, o2_ref, xc_ref):
    xc_ref[0:64, :] = f1_ref[0].astype(jnp.bfloat16)
    xc_ref[64:128, :] = f2_ref[0].astype(jnp.bfloat16)
    xc_ref[128:256, :] = f3_ref[0].astype(jnp.bfloat16)
    xc_ref[256:512, :] = f4_ref[0].astype(jnp.bfloat16)
    y = jax.lax.dot_general(w5_ref[...], xc_ref[...],
                            (((1,), (0,)), ((), ())),
                            preferred_element_type=jnp.float32)
    z = _lrelu(y / _RS * g5_ref[...] + b5_ref[...])
    o1_ref[0, 0] = jnp.max(z, axis=1)
    o2_ref[0, 0] = jnp.sum(z, axis=1) * (1.0 / _N)


def _final_conv(f1, f2, f3, f4, w5b, g5, b5):
    return pl.pallas_call(
        _final_conv_body,
        grid=(_B,),
        in_specs=[
            pl.BlockSpec((1, 64, _N), lambda b: (b, 0, 0)),
            pl.BlockSpec((1, 64, _N), lambda b: (b, 0, 0)),
            pl.BlockSpec((1, 128, _N), lambda b: (b, 0, 0)),
            pl.BlockSpec((1, 256, _N), lambda b: (b, 0, 0)),
            pl.BlockSpec((256, 512), lambda b: (0, 0)),
            pl.BlockSpec((256, 1), lambda b: (0, 0)),
            pl.BlockSpec((256, 1), lambda b: (0, 0)),
        ],
        out_specs=[
            pl.BlockSpec((1, 1, 256), lambda b: (b, 0, 0)),
            pl.BlockSpec((1, 1, 256), lambda b: (b, 0, 0)),
        ],
        out_shape=[
            jax.ShapeDtypeStruct((_B, 1, 256), jnp.float32),
            jax.ShapeDtypeStruct((_B, 1, 256), jnp.float32),
        ],
        scratch_shapes=[
            pltpu.VMEM((512, _N), jnp.bfloat16),
        ],
        compiler_params=pltpu.CompilerParams(
            dimension_semantics=("arbitrary",)),
    )(f1, f2, f3, f4, w5b, g5, b5)


def _head_body(x1_ref, x2_ref, w1_ref, g6_ref, b6_ref, w2_ref, b2_ref,
               g7_ref, b7_ref, w3_ref, b3_ref, o_ref):
    h = jnp.concatenate([x1_ref[:, 0, :], x2_ref[:, 0, :]], axis=1)
    y = jax.lax.dot_general(h.astype(jnp.bfloat16), w1_ref[...],
                            (((1,), (1,)), ((), ())),
                            preferred_element_type=jnp.float32)
    h = _lrelu(y / _RS * g6_ref[...] + b6_ref[...])
    y = jax.lax.dot_general(h.astype(jnp.bfloat16), w2_ref[...],
                            (((1,), (1,)), ((), ())),
                            preferred_element_type=jnp.float32) + b2_ref[...]
    h = _lrelu(y / _RS * g7_ref[...] + b7_ref[...])
    o_ref[...] = jax.lax.dot_general(h.astype(jnp.bfloat16), w3_ref[...],
                                     (((1,), (1,)), ((), ())),
                                     preferred_element_type=jnp.float32) \
        + b3_ref[...]


def _head(x1, x2, w1b, g6, b6, w2b, b2, g7, b7, w3b, b3):
    return pl.pallas_call(
        _head_body,
        out_shape=jax.ShapeDtypeStruct((_B, 3), jnp.float32),
    )(x1, x2, w1b, g6, b6, w2b, b2, g7, b7, w3b, b3)


def kernel(x, convW0, bn_g0, bn_b0, convW1, bn_g1, bn_b1, convW2, bn_g2,
           bn_b2, convW3, bn_g3, bn_b3, W5, bn5_g, bn5_b, W1, bn6_g, bn6_b,
           W2, b2, bn7_g, bn7_b, W3, b3):
    mlp = [3, 64, 64, 128, 256]
    convs = [(convW0, bn_g0, bn_b0), (convW1, bn_g1, bn_b1),
             (convW2, bn_g2, bn_b2), (convW3, bn_g3, bn_b3)]

    cur = x
    feats = []
    for i, (cw, g, b) in enumerate(convs):
        cin, cout = mlp[i], mlp[i + 1]
        cpad = max(8, cin)
        wb = cw.astype(jnp.bfloat16)
        if cpad != cin:
            wbp = jnp.zeros((cout, 2 * cpad), jnp.bfloat16)
            wbp = wbp.at[:, :2 * cin].set(wb)
        else:
            wbp = wb
        cur = _edge_layer(cur, wbp, g[:, None], b[:, None], cin, cout, cpad)
        feats.append(cur)

    x1, x2 = _final_conv(feats[0], feats[1], feats[2], feats[3],
                         W5.astype(jnp.bfloat16), bn5_g[:, None],
                         bn5_b[:, None])

    return _head(x1, x2, W1.astype(jnp.bfloat16), bn6_g[None, :],
                 bn6_b[None, :], W2.astype(jnp.bfloat16), b2[None, :],
                 bn7_g[None, :], bn7_b[None, :], W3.astype(jnp.bfloat16),
                 b3[None, :])


# speedup vs baseline: 13.2703x; 1.1305x over previous
import functools
import jax
import jax.numpy as jnp
import numpy as np
from jax.experimental import pallas as pl
from jax.experimental.pallas import tpu as pltpu

_N = 1024
_B = 32
_K = 20
_RS = np.sqrt(np.float32(1.0 + 1e-5), dtype=np.float32)


def _lrelu(v):
    return jnp.where(v > 0, v, 0.2 * v)


def _gather_cols(src_ref, rows, jstar, out_dtype=None):
    lo = jstar & 127
    hi = jstar >> 7
    parts = []
    for a in range(8):
        sl = slice(a * 128, (a + 1) * 128)
        lo_a = jnp.broadcast_to(lo[None, sl], (rows, 128))
        hi_a = hi[None, sl]
        acc = None
        for t in range(8):
            src = src_ref[:, t * 128:(t + 1) * 128]
            gt = jnp.take_along_axis(src, lo_a, axis=1,
                                     mode="promise_in_bounds")
            acc = gt if acc is None else jnp.where(hi_a == t, gt, acc)
        parts.append(acc)
    return jnp.concatenate(parts, axis=1)


def _edge_layer_body(x_ref, wb_ref, g_ref, b_ref, o_ref, pd_ref, gf_ref,
                     mx_ref, mn_ref, *, cin, cout, cpad):
    x = x_ref[0]
    xb = x.astype(jnp.bfloat16)
    g = jax.lax.dot_general(xb, xb, (((0,), (0,)), ((), ())),
                            preferred_element_type=jnp.float32)
    inner = -2.0 * g
    sq = jnp.sum(x * x, axis=0, keepdims=True)
    sq_col = jnp.transpose(sq)
    pd_ref[...] = ((-sq) - inner) - sq_col

    if cpad == cin:
        gf_ref[cin:, :] = xb

    mx_ref[...] = jnp.full((cout, _N), -jnp.inf, jnp.float32)
    mn_ref[...] = jnp.full((cout, _N), jnp.inf, jnp.float32)

    def body(_, jprev):
        jidx = jax.lax.broadcasted_iota(jnp.int32, (_N, _N), 0)
        kk = jnp.where(jidx == jprev[None, :], -jnp.inf, pd_ref[...])
        pd_ref[...] = kk
        m = jnp.max(kk, axis=0)
        jstar = jnp.min(jnp.where(kk == m[None, :], jidx, _N), axis=0)
        xg = _gather_cols(x_ref.at[0], cin, jstar)
        fk = (xg - x).astype(jnp.bfloat16)
        if cpad == cin:
            gf_ref[:cin, :] = fk
            gf = gf_ref[...]
        else:
            gf = jnp.concatenate(
                [fk, xb,
                 jnp.zeros((2 * (cpad - cin), _N), jnp.bfloat16)], axis=0)
        yk = jax.lax.dot_general(wb_ref[...], gf,
                                 (((1,), (0,)), ((), ())),
                                 preferred_element_type=jnp.float32)
        mx_ref[...] = jnp.maximum(mx_ref[...], yk)
        mn_ref[...] = jnp.minimum(mn_ref[...], yk)
        return jstar

    jax.lax.fori_loop(0, _K, body, jnp.full((_N,), -1, jnp.int32))

    gv = g_ref[...]
    sel = jnp.where(gv >= 0, mx_ref[...], mn_ref[...])
    o_ref[0] = _lrelu(sel / _RS * gv + b_ref[...])


def _edge_layer(x, wb, g, b, cin, cout, cpad):
    body = functools.partial(_edge_layer_body, cin=cin, cout=cout, cpad=cpad)
    return pl.pallas_call(
        body,
        grid=(_B,),
        in_specs=[
            pl.BlockSpec((1, cin, _N), lambda bb: (bb, 0, 0)),
            pl.BlockSpec((cout, 2 * cpad), lambda bb: (0, 0)),
            pl.BlockSpec((cout, 1), lambda bb: (0, 0)),
            pl.BlockSpec((cout, 1), lambda bb: (0, 0)),
        ],
        out_specs=pl.BlockSpec((1, cout, _N), lambda bb: (bb, 0, 0)),
        out_shape=jax.ShapeDtypeStruct((_B, cout, _N), jnp.float32),
        scratch_shapes=[
            pltpu.VMEM((_N, _N), jnp.float32),
            pltpu.VMEM((2 * cpad, _N), jnp.bfloat16),
            pltpu.VMEM((cout, _N), jnp.float32),
            pltpu.VMEM((cout, _N), jnp.float32),
        ],
        compiler_params=pltpu.CompilerParams(
            dimension_semantics=("arbitrary",)),
    )(x, wb, g, b)


def _final_conv_body(f1_ref, f2_ref, f3_ref, f4_ref, w5_ref, g5_ref, b5_ref,
                     o1_ref, o2_ref, xc_ref):
    xc_ref[0:64, :] = f1_ref[0].astype(jnp.bfloat16)
    xc_ref[64:128, :] = f2_ref[0].astype(jnp.bfloat16)
    xc_ref[128:256, :] = f3_ref[0].astype(jnp.bfloat16)
    xc_ref[256:512, :] = f4_ref[0].astype(jnp.bfloat16)
    y = jax.lax.dot_general(w5_ref[...], xc_ref[...],
                            (((1,), (0,)), ((), ())),
                            preferred_element_type=jnp.float32)
    z = _lrelu(y / _RS * g5_ref[...] + b5_ref[...])
    o1_ref[0, 0] = jnp.max(z, axis=1)
    o2_ref[0, 0] = jnp.sum(z, axis=1) * (1.0 / _N)


def _final_conv(f1, f2, f3, f4, w5b, g5, b5):
    return pl.pallas_call(
        _final_conv_body,
        grid=(_B,),
        in_specs=[
            pl.BlockSpec((1, 64, _N), lambda b: (b, 0, 0)),
            pl.BlockSpec((1, 64, _N), lambda b: (b, 0, 0)),
            pl.BlockSpec((1, 128, _N), lambda b: (b, 0, 0)),
            pl.BlockSpec((1, 256, _N), lambda b: (b, 0, 0)),
            pl.BlockSpec((256, 512), lambda b: (0, 0)),
            pl.BlockSpec((256, 1), lambda b: (0, 0)),
            pl.BlockSpec((256, 1), lambda b: (0, 0)),
        ],
        out_specs=[
            pl.BlockSpec((1, 1, 256), lambda b: (b, 0, 0)),
            pl.BlockSpec((1, 1, 256), lambda b: (b, 0, 0)),
        ],
        out_shape=[
            jax.ShapeDtypeStruct((_B, 1, 256), jnp.float32),
            jax.ShapeDtypeStruct((_B, 1, 256), jnp.float32),
        ],
        scratch_shapes=[
            pltpu.VMEM((512, _N), jnp.bfloat16),
        ],
        compiler_params=pltpu.CompilerParams(
            dimension_semantics=("arbitrary",)),
    )(f1, f2, f3, f4, w5b, g5, b5)


def _head_body(x1_ref, x2_ref, w1_ref, g6_ref, b6_ref, w2_ref, b2_ref,
               g7_ref, b7_ref, w3_ref, b3_ref, o_ref):
    h = jnp.concatenate([x1_ref[:, 0, :], x2_ref[:, 0, :]], axis=1)
    y = jax.lax.dot_general(h.astype(jnp.bfloat16), w1_ref[...],
                            (((1,), (1,)), ((), ())),
                            preferred_element_type=jnp.float32)
    h = _lrelu(y / _RS * g6_ref[...] + b6_ref[...])
    y = jax.lax.dot_general(h.astype(jnp.bfloat16), w2_ref[...],
                            (((1,), (1,)), ((), ())),
                            preferred_element_type=jnp.float32) + b2_ref[...]
    h = _lrelu(y / _RS * g7_ref[...] + b7_ref[...])
    o_ref[...] = jax.lax.dot_general(h.astype(jnp.bfloat16), w3_ref[...],
                                     (((1,), (1,)), ((), ())),
                                     preferred_element_type=jnp.float32) \
        + b3_ref[...]


def _head(x1, x2, w1b, g6, b6, w2b, b2, g7, b7, w3b, b3):
    return pl.pallas_call(
        _head_body,
        out_shape=jax.ShapeDtypeStruct((_B, 3), jnp.float32),
    )(x1, x2, w1b, g6, b6, w2b, b2, g7, b7, w3b, b3)


def kernel(x, convW0, bn_g0, bn_b0, convW1, bn_g1, bn_b1, convW2, bn_g2,
           bn_b2, convW3, bn_g3, bn_b3, W5, bn5_g, bn5_b, W1, bn6_g, bn6_b,
           W2, b2, bn7_g, bn7_b, W3, b3):
    mlp = [3, 64, 64, 128, 256]
    convs = [(convW0, bn_g0, bn_b0), (convW1, bn_g1, bn_b1),
             (convW2, bn_g2, bn_b2), (convW3, bn_g3, bn_b3)]

    cur = x
    feats = []
    for i, (cw, g, b) in enumerate(convs):
        cin, cout = mlp[i], mlp[i + 1]
        cpad = max(8, cin)
        wb = cw.astype(jnp.bfloat16)
        if cpad != cin:
            wbp = jnp.zeros((cout, 2 * cpad), jnp.bfloat16)
            wbp = wbp.at[:, :2 * cin].set(wb)
        else:
            wbp = wb
        cur = _edge_layer(cur, wbp, g[:, None], b[:, None], cin, cout, cpad)
        feats.append(cur)

    x1, x2 = _final_conv(feats[0], feats[1], feats[2], feats[3],
                         W5.astype(jnp.bfloat16), bn5_g[:, None],
                         bn5_b[:, None])

    return _head(x1, x2, W1.astype(jnp.bfloat16), bn6_g[None, :],
                 bn6_b[None, :], W2.astype(jnp.bfloat16), b2[None, :],
                 bn7_g[None, :], bn7_b[None, :], W3.astype(jnp.bfloat16),
                 b3[None, :])


# speedup vs baseline: 14.6307x; 1.1025x over previous
import functools
import jax
import jax.numpy as jnp
import numpy as np
from jax.experimental import pallas as pl
from jax.experimental.pallas import tpu as pltpu

_N = 1024
_B = 32
_K = 20
_RS = np.sqrt(np.float32(1.0 + 1e-5), dtype=np.float32)


def _lrelu(v):
    return jnp.where(v > 0, v, 0.2 * v)


def _gather_cols(src_ref, rows, jstar, out_dtype=None):
    lo = jstar & 127
    hi = jstar >> 7
    parts = []
    for a in range(8):
        sl = slice(a * 128, (a + 1) * 128)
        lo_a = jnp.broadcast_to(lo[None, sl], (rows, 128))
        hi_a = hi[None, sl]
        acc = None
        for t in range(8):
            src = src_ref[:, t * 128:(t + 1) * 128]
            gt = jnp.take_along_axis(src, lo_a, axis=1,
                                     mode="promise_in_bounds")
            acc = gt if acc is None else jnp.where(hi_a == t, gt, acc)
        parts.append(acc)
    return jnp.concatenate(parts, axis=1)


def _edge_layer_body(x_ref, wb_ref, g_ref, b_ref, o_ref, pd_ref, gf_ref,
                     mx_ref, mn_ref, *, cin, cout, cpad):
    x = x_ref[0]
    xb = x.astype(jnp.bfloat16)
    g = jax.lax.dot_general(xb, xb, (((0,), (0,)), ((), ())),
                            preferred_element_type=jnp.float32)
    inner = -2.0 * g
    sq = jnp.sum(x * x, axis=0, keepdims=True)
    sq_col = jnp.transpose(sq)
    pd_ref[...] = ((-sq) - inner) - sq_col

    if cpad == cin:
        gf_ref[cin:, :] = xb

    mx_ref[...] = jnp.full((cout, _N), -jnp.inf, jnp.float32)
    mn_ref[...] = jnp.full((cout, _N), jnp.inf, jnp.float32)

    def body(_, jprev):
        jidx = jax.lax.broadcasted_iota(jnp.int32, (_N, _N), 0)
        kk = jnp.where(jidx == jprev[None, :], -jnp.inf, pd_ref[...])
        pd_ref[...] = kk
        jstar = jnp.argmax(kk, axis=0).astype(jnp.int32)
        xg = _gather_cols(x_ref.at[0], cin, jstar)
        fk = (xg - x).astype(jnp.bfloat16)
        if cpad == cin:
            gf_ref[:cin, :] = fk
            gf = gf_ref[...]
        else:
            gf = jnp.concatenate(
                [fk, xb,
                 jnp.zeros((2 * (cpad - cin), _N), jnp.bfloat16)], axis=0)
        yk = jax.lax.dot_general(wb_ref[...], gf,
                                 (((1,), (0,)), ((), ())),
                                 preferred_element_type=jnp.float32)
        mx_ref[...] = jnp.maximum(mx_ref[...], yk)
        mn_ref[...] = jnp.minimum(mn_ref[...], yk)
        return jstar

    jax.lax.fori_loop(0, _K, body, jnp.full((_N,), -1, jnp.int32))

    gv = g_ref[...]
    sel = jnp.where(gv >= 0, mx_ref[...], mn_ref[...])
    o_ref[0] = _lrelu(sel / _RS * gv + b_ref[...])


def _edge_layer(x, wb, g, b, cin, cout, cpad):
    body = functools.partial(_edge_layer_body, cin=cin, cout=cout, cpad=cpad)
    return pl.pallas_call(
        body,
        grid=(_B,),
        in_specs=[
            pl.BlockSpec((1, cin, _N), lambda bb: (bb, 0, 0)),
            pl.BlockSpec((cout, 2 * cpad), lambda bb: (0, 0)),
            pl.BlockSpec((cout, 1), lambda bb: (0, 0)),
            pl.BlockSpec((cout, 1), lambda bb: (0, 0)),
        ],
        out_specs=pl.BlockSpec((1, cout, _N), lambda bb: (bb, 0, 0)),
        out_shape=jax.ShapeDtypeStruct((_B, cout, _N), jnp.float32),
        scratch_shapes=[
            pltpu.VMEM((_N, _N), jnp.float32),
            pltpu.VMEM((2 * cpad, _N), jnp.bfloat16),
            pltpu.VMEM((cout, _N), jnp.float32),
            pltpu.VMEM((cout, _N), jnp.float32),
        ],
        compiler_params=pltpu.CompilerParams(
            dimension_semantics=("arbitrary",)),
    )(x, wb, g, b)


def _final_conv_body(f1_ref, f2_ref, f3_ref, f4_ref, w5_ref, g5_ref, b5_ref,
                     o1_ref, o2_ref, xc_ref):
    xc_ref[0:64, :] = f1_ref[0].astype(jnp.bfloat16)
    xc_ref[64:128, :] = f2_ref[0].astype(jnp.bfloat16)
    xc_ref[128:256, :] = f3_ref[0].astype(jnp.bfloat16)
    xc_ref[256:512, :] = f4_ref[0].astype(jnp.bfloat16)
    y = jax.lax.dot_general(w5_ref[...], xc_ref[...],
                            (((1,), (0,)), ((), ())),
                            preferred_element_type=jnp.float32)
    z = _lrelu(y / _RS * g5_ref[...] + b5_ref[...])
    o1_ref[0, 0] = jnp.max(z, axis=1)
    o2_ref[0, 0] = jnp.sum(z, axis=1) * (1.0 / _N)


def _final_conv(f1, f2, f3, f4, w5b, g5, b5):
    return pl.pallas_call(
        _final_conv_body,
        grid=(_B,),
        in_specs=[
            pl.BlockSpec((1, 64, _N), lambda b: (b, 0, 0)),
            pl.BlockSpec((1, 64, _N), lambda b: (b, 0, 0)),
            pl.BlockSpec((1, 128, _N), lambda b: (b, 0, 0)),
            pl.BlockSpec((1, 256, _N), lambda b: (b, 0, 0)),
            pl.BlockSpec((256, 512), lambda b: (0, 0)),
            pl.BlockSpec((256, 1), lambda b: (0, 0)),
            pl.BlockSpec((256, 1), lambda b: (0, 0)),
        ],
        out_specs=[
            pl.BlockSpec((1, 1, 256), lambda b: (b, 0, 0)),
            pl.BlockSpec((1, 1, 256), lambda b: (b, 0, 0)),
        ],
        out_shape=[
            jax.ShapeDtypeStruct((_B, 1, 256), jnp.float32),
            jax.ShapeDtypeStruct((_B, 1, 256), jnp.float32),
        ],
        scratch_shapes=[
            pltpu.VMEM((512, _N), jnp.bfloat16),
        ],
        compiler_params=pltpu.CompilerParams(
            dimension_semantics=("arbitrary",)),
    )(f1, f2, f3, f4, w5b, g5, b5)


def _head_body(x1_ref, x2_ref, w1_ref, g6_ref, b6_ref, w2_ref, b2_ref,
               g7_ref, b7_ref, w3_ref, b3_ref, o_ref):
    h = jnp.concatenate([x1_ref[:, 0, :], x2_ref[:, 0, :]], axis=1)
    y = jax.lax.dot_general(h.astype(jnp.bfloat16), w1_ref[...],
                            (((1,), (1,)), ((), ())),
                            preferred_element_type=jnp.float32)
    h = _lrelu(y / _RS * g6_ref[...] + b6_ref[...])
    y = jax.lax.dot_general(h.astype(jnp.bfloat16), w2_ref[...],
                            (((1,), (1,)), ((), ())),
                            preferred_element_type=jnp.float32) + b2_ref[...]
    h = _lrelu(y / _RS * g7_ref[...] + b7_ref[...])
    o_ref[...] = jax.lax.dot_general(h.astype(jnp.bfloat16), w3_ref[...],
                                     (((1,), (1,)), ((), ())),
                                     preferred_element_type=jnp.float32) \
        + b3_ref[...]


def _head(x1, x2, w1b, g6, b6, w2b, b2, g7, b7, w3b, b3):
    return pl.pallas_call(
        _head_body,
        out_shape=jax.ShapeDtypeStruct((_B, 3), jnp.float32),
    )(x1, x2, w1b, g6, b6, w2b, b2, g7, b7, w3b, b3)


def kernel(x, convW0, bn_g0, bn_b0, convW1, bn_g1, bn_b1, convW2, bn_g2,
           bn_b2, convW3, bn_g3, bn_b3, W5, bn5_g, bn5_b, W1, bn6_g, bn6_b,
           W2, b2, bn7_g, bn7_b, W3, b3):
    mlp = [3, 64, 64, 128, 256]
    convs = [(convW0, bn_g0, bn_b0), (convW1, bn_g1, bn_b1),
             (convW2, bn_g2, bn_b2), (convW3, bn_g3, bn_b3)]

    cur = x
    feats = []
    for i, (cw, g, b) in enumerate(convs):
        cin, cout = mlp[i], mlp[i + 1]
        cpad = max(8, cin)
        wb = cw.astype(jnp.bfloat16)
        if cpad != cin:
            wbp = jnp.zeros((cout, 2 * cpad), jnp.bfloat16)
            wbp = wbp.at[:, :2 * cin].set(wb)
        else:
            wbp = wb
        cur = _edge_layer(cur, wbp, g[:, None], b[:, None], cin, cout, cpad)
        feats.append(cur)

    x1, x2 = _final_conv(feats[0], feats[1], feats[2], feats[3],
                         W5.astype(jnp.bfloat16), bn5_g[:, None],
                         bn5_b[:, None])

    return _head(x1, x2, W1.astype(jnp.bfloat16), bn6_g[None, :],
                 bn6_b[None, :], W2.astype(jnp.bfloat16), b2[None, :],
                 bn7_g[None, :], bn7_b[None, :], W3.astype(jnp.bfloat16),
                 b3[None, :])


# speedup vs baseline: 14.6670x; 1.0025x over previous
import functools
import jax
import jax.numpy as jnp
import numpy as np
from jax.experimental import pallas as pl
from jax.experimental.pallas import tpu as pltpu

_N = 1024
_B = 32
_K = 20
_RS = np.sqrt(np.float32(1.0 + 1e-5), dtype=np.float32)


def _lrelu(v):
    return jnp.where(v > 0, v, 0.2 * v)


def _gather_cols(src_ref, rows, jstar, out_dtype=None):
    lo = jstar & 127
    hi = jstar >> 7
    parts = []
    for a in range(8):
        sl = slice(a * 128, (a + 1) * 128)
        lo_a = jnp.broadcast_to(lo[None, sl], (rows, 128))
        hi_a = hi[None, sl]
        acc = None
        for t in range(8):
            src = src_ref[:, t * 128:(t + 1) * 128]
            gt = jnp.take_along_axis(src, lo_a, axis=1,
                                     mode="promise_in_bounds")
            acc = gt if acc is None else jnp.where(hi_a == t, gt, acc)
        parts.append(acc)
    return jnp.concatenate(parts, axis=1)


def _edge_layer_body(x_ref, wb_ref, ag_ref, b_ref, o_ref, pd_ref, gf_ref,
                     mx_ref, *, cin, cout, cpad):
    x = x_ref[0]
    xb = x.astype(jnp.bfloat16)
    g = jax.lax.dot_general(xb, xb, (((0,), (0,)), ((), ())),
                            preferred_element_type=jnp.float32)
    inner = -2.0 * g
    sq = jnp.sum(x * x, axis=0, keepdims=True)
    sq_col = jnp.transpose(sq)
    pd_ref[...] = ((-sq) - inner) - sq_col

    if cpad == cin:
        gf_ref[cin:, :] = xb

    mx_ref[...] = jnp.full((cout, _N), -jnp.inf, jnp.float32)

    def body(_, jprev):
        jidx = jax.lax.broadcasted_iota(jnp.int32, (_N, _N), 0)
        kk = jnp.where(jidx == jprev[None, :], -jnp.inf, pd_ref[...])
        pd_ref[...] = kk
        jstar = jnp.argmax(kk, axis=0).astype(jnp.int32)
        xg = _gather_cols(x_ref.at[0], cin, jstar)
        fk = (xg - x).astype(jnp.bfloat16)
        if cpad == cin:
            gf_ref[:cin, :] = fk
            gf = gf_ref[...]
        else:
            gf = jnp.concatenate(
                [fk, xb,
                 jnp.zeros((2 * (cpad - cin), _N), jnp.bfloat16)], axis=0)
        yk = jax.lax.dot_general(wb_ref[...], gf,
                                 (((1,), (0,)), ((), ())),
                                 preferred_element_type=jnp.float32)
        mx_ref[...] = jnp.maximum(mx_ref[...], yk)
        return jstar

    jax.lax.fori_loop(0, _K, body, jnp.full((_N,), -1, jnp.int32))

    o_ref[0] = _lrelu(mx_ref[...] / _RS * ag_ref[...] + b_ref[...])


def _edge_layer(x, wb, ag, b, cin, cout, cpad):
    body = functools.partial(_edge_layer_body, cin=cin, cout=cout, cpad=cpad)
    return pl.pallas_call(
        body,
        grid=(_B,),
        in_specs=[
            pl.BlockSpec((1, cin, _N), lambda bb: (bb, 0, 0)),
            pl.BlockSpec((cout, 2 * cpad), lambda bb: (0, 0)),
            pl.BlockSpec((cout, 1), lambda bb: (0, 0)),
            pl.BlockSpec((cout, 1), lambda bb: (0, 0)),
        ],
        out_specs=pl.BlockSpec((1, cout, _N), lambda bb: (bb, 0, 0)),
        out_shape=jax.ShapeDtypeStruct((_B, cout, _N), jnp.float32),
        scratch_shapes=[
            pltpu.VMEM((_N, _N), jnp.float32),
            pltpu.VMEM((2 * cpad, _N), jnp.bfloat16),
            pltpu.VMEM((cout, _N), jnp.float32),
        ],
        compiler_params=pltpu.CompilerParams(
            dimension_semantics=("arbitrary",)),
    )(x, wb, ag, b)


def _final_conv_body(f1_ref, f2_ref, f3_ref, f4_ref, w5_ref, g5_ref, b5_ref,
                     o1_ref, o2_ref, xc_ref):
    xc_ref[0:64, :] = f1_ref[0].astype(jnp.bfloat16)
    xc_ref[64:128, :] = f2_ref[0].astype(jnp.bfloat16)
    xc_ref[128:256, :] = f3_ref[0].astype(jnp.bfloat16)
    xc_ref[256:512, :] = f4_ref[0].astype(jnp.bfloat16)
    y = jax.lax.dot_general(w5_ref[...], xc_ref[...],
                            (((1,), (0,)), ((), ())),
                            preferred_element_type=jnp.float32)
    z = _lrelu(y / _RS * g5_ref[...] + b5_ref[...])
    o1_ref[0, 0] = jnp.max(z, axis=1)
    o2_ref[0, 0] = jnp.sum(z, axis=1) * (1.0 / _N)


def _final_conv(f1, f2, f3, f4, w5b, g5, b5):
    return pl.pallas_call(
        _final_conv_body,
        grid=(_B,),
        in_specs=[
            pl.BlockSpec((1, 64, _N), lambda b: (b, 0, 0)),
            pl.BlockSpec((1, 64, _N), lambda b: (b, 0, 0)),
            pl.BlockSpec((1, 128, _N), lambda b: (b, 0, 0)),
            pl.BlockSpec((1, 256, _N), lambda b: (b, 0, 0)),
            pl.BlockSpec((256, 512), lambda b: (0, 0)),
            pl.BlockSpec((256, 1), lambda b: (0, 0)),
            pl.BlockSpec((256, 1), lambda b: (0, 0)),
        ],
        out_specs=[
            pl.BlockSpec((1, 1, 256), lambda b: (b, 0, 0)),
            pl.BlockSpec((1, 1, 256), lambda b: (b, 0, 0)),
        ],
        out_shape=[
            jax.ShapeDtypeStruct((_B, 1, 256), jnp.float32),
            jax.ShapeDtypeStruct((_B, 1, 256), jnp.float32),
        ],
        scratch_shapes=[
            pltpu.VMEM((512, _N), jnp.bfloat16),
        ],
        compiler_params=pltpu.CompilerParams(
            dimension_semantics=("arbitrary",)),
    )(f1, f2, f3, f4, w5b, g5, b5)


def _head_body(x1_ref, x2_ref, w1_ref, g6_ref, b6_ref, w2_ref, b2_ref,
               g7_ref, b7_ref, w3_ref, b3_ref, o_ref):
    h = jnp.concatenate([x1_ref[:, 0, :], x2_ref[:, 0, :]], axis=1)
    y = jax.lax.dot_general(h.astype(jnp.bfloat16), w1_ref[...],
                            (((1,), (1,)), ((), ())),
                            preferred_element_type=jnp.float32)
    h = _lrelu(y / _RS * g6_ref[...] + b6_ref[...])
    y = jax.lax.dot_general(h.astype(jnp.bfloat16), w2_ref[...],
                            (((1,), (1,)), ((), ())),
                            preferred_element_type=jnp.float32) + b2_ref[...]
    h = _lrelu(y / _RS * g7_ref[...] + b7_ref[...])
    o_ref[...] = jax.lax.dot_general(h.astype(jnp.bfloat16), w3_ref[...],
                                     (((1,), (1,)), ((), ())),
                                     preferred_element_type=jnp.float32) \
        + b3_ref[...]


def _head(x1, x2, w1b, g6, b6, w2b, b2, g7, b7, w3b, b3):
    return pl.pallas_call(
        _head_body,
        out_shape=jax.ShapeDtypeStruct((_B, 3), jnp.float32),
    )(x1, x2, w1b, g6, b6, w2b, b2, g7, b7, w3b, b3)


def kernel(x, convW0, bn_g0, bn_b0, convW1, bn_g1, bn_b1, convW2, bn_g2,
           bn_b2, convW3, bn_g3, bn_b3, W5, bn5_g, bn5_b, W1, bn6_g, bn6_b,
           W2, b2, bn7_g, bn7_b, W3, b3):
    mlp = [3, 64, 64, 128, 256]
    convs = [(convW0, bn_g0, bn_b0), (convW1, bn_g1, bn_b1),
             (convW2, bn_g2, bn_b2), (convW3, bn_g3, bn_b3)]

    cur = x
    feats = []
    for i, (cw, g, b) in enumerate(convs):
        cin, cout = mlp[i], mlp[i + 1]
        cpad = max(8, cin)
        sg = jnp.where(g >= 0, 1.0, -1.0)
        wb = (cw * sg[:, None]).astype(jnp.bfloat16)
        if cpad != cin:
            wbp = jnp.zeros((cout, 2 * cpad), jnp.bfloat16)
            wbp = wbp.at[:, :2 * cin].set(wb)
        else:
            wbp = wb
        cur = _edge_layer(cur, wbp, (sg * g)[:, None], b[:, None], cin,
                          cout, cpad)
        feats.append(cur)

    x1, x2 = _final_conv(feats[0], feats[1], feats[2], feats[3],
                         W5.astype(jnp.bfloat16), bn5_g[:, None],
                         bn5_b[:, None])

    return _head(x1, x2, W1.astype(jnp.bfloat16), bn6_g[None, :],
                 bn6_b[None, :], W2.astype(jnp.bfloat16), b2[None, :],
                 bn7_g[None, :], bn7_b[None, :], W3.astype(jnp.bfloat16),
                 b3[None, :])
